# Initial kernel scaffold; baseline (speedup 1.0000x reference)
import jax
import jax.numpy as jnp
from jax.experimental import pallas as pl


def kernel(x, edge_index, W1_l, W1_r, b1, W2_l, W2_r, b2):
    raise NotImplementedError("write your pallas kernel here")


# speedup vs baseline: 6.8387x; 6.8387x over previous
import functools

import jax
import jax.numpy as jnp
from jax import lax
from jax.experimental import pallas as pl
from jax.experimental.pallas import tpu as pltpu
from jax.experimental.pallas import tpu_sc as plsc

N = 10000
E = 320000
D = 128

NC = 2
NS = 16
L = 16
NW = NC * NS
EPT = E // NW
CH = 80
NCHUNK = EPT // CH
RA = 624
TAIL = N - NS * RA
ZR = 16

BR = 400

_mesh = plsc.VectorSubcoreMesh(core_axis_name="c", subcore_axis_name="s")


@functools.partial(
    pl.kernel,
    mesh=_mesh,
    out_type=jax.ShapeDtypeStruct((NC, N, D), jnp.float32),
    scratch_types=[
        pltpu.VMEM((NCHUNK, CH), jnp.int32),
        pltpu.VMEM((NCHUNK, CH), jnp.int32),
        pltpu.VMEM((CH, D), jnp.float32),
        pltpu.VMEM((ZR, D), jnp.float32),
        pltpu.VMEM_SHARED((N, D), jnp.float32),
    ],
)
def _sc_agg(feat_hbm, src_hbm, dst_hbm, agg_out, sidx, didx, rows, zbuf,
            acc_sh):
    cid = lax.axis_index("c")
    sid = lax.axis_index("s")
    wid = cid * NS + sid
    rbase = sid * RA
    is_last = sid == NS - 1

    pltpu.sync_copy(src_hbm.at[wid], sidx)
    pltpu.sync_copy(dst_hbm.at[wid], didx)

    @pl.loop(0, ZR)
    def _(r):
        @pl.loop(0, D, step=L)
        def _(c0):
            zbuf[r, pl.ds(c0, L)] = jnp.zeros((L,), jnp.float32)

    @pl.loop(0, RA, step=ZR)
    def _(r0):
        pltpu.sync_copy(zbuf, acc_sh.at[pl.ds(rbase + r0, ZR)])

    @pl.when(is_last)
    def _():
        pltpu.sync_copy(zbuf.at[pl.ds(0, TAIL)],
                        acc_sh.at[pl.ds(NS * RA, TAIL)])

    plsc.subcore_barrier()

    @pl.loop(0, NCHUNK)
    def _(c):
        pltpu.sync_copy(feat_hbm.at[sidx.at[c]], rows)
        pltpu.sync_copy(rows, acc_sh.at[didx.at[c]], add=True)

    plsc.subcore_barrier()

    pltpu.sync_copy(acc_sh.at[pl.ds(rbase, RA)],
                    agg_out.at[cid, pl.ds(rbase, RA)])

    @pl.when(is_last)
    def _():
        pltpu.sync_copy(acc_sh.at[pl.ds(NS * RA, TAIL)],
                        agg_out.at[cid, pl.ds(NS * RA, TAIL)])


@functools.partial(
    pl.kernel,
    mesh=_mesh,
    out_type=jax.ShapeDtypeStruct((NC, N, D), jnp.float32),
    scratch_types=[
        pltpu.VMEM((NCHUNK, CH), jnp.int32),
        pltpu.VMEM((CH, D), jnp.float32),
        pltpu.VMEM((ZR, D), jnp.float32),
        pltpu.VMEM_SHARED((N, D), jnp.float32),
    ],
)
def _sc_deg(dst_hbm, deg_out, didx, ones, dzbuf, deg_sh):
    cid = lax.axis_index("c")
    sid = lax.axis_index("s")
    wid = cid * NS + sid
    rbase = sid * RA
    is_last = sid == NS - 1

    pltpu.sync_copy(dst_hbm.at[wid], didx)

    @pl.loop(0, ZR)
    def _(r):
        @pl.loop(0, D, step=L)
        def _(c0):
            dzbuf[r, pl.ds(c0, L)] = jnp.zeros((L,), jnp.float32)

    @pl.loop(0, CH)
    def _(r):
        @pl.loop(0, D, step=L)
        def _(c0):
            ones[r, pl.ds(c0, L)] = jnp.ones((L,), jnp.float32)

    @pl.loop(0, RA, step=ZR)
    def _(r0):
        pltpu.sync_copy(dzbuf, deg_sh.at[pl.ds(rbase + r0, ZR)])

    @pl.when(is_last)
    def _():
        pltpu.sync_copy(dzbuf.at[pl.ds(0, TAIL)],
                        deg_sh.at[pl.ds(NS * RA, TAIL)])

    plsc.subcore_barrier()

    @pl.loop(0, NCHUNK)
    def _(c):
        pltpu.sync_copy(ones, deg_sh.at[didx.at[c]], add=True)

    plsc.subcore_barrier()

    pltpu.sync_copy(deg_sh.at[pl.ds(rbase, RA)],
                    deg_out.at[cid, pl.ds(rbase, RA)])

    @pl.when(is_last)
    def _():
        pltpu.sync_copy(deg_sh.at[pl.ds(NS * RA, TAIL)],
                        deg_out.at[cid, pl.ds(NS * RA, TAIL)])


def _tc_self(x, w, b):
    def body(x_ref, w_ref, b_ref, o_ref):
        o_ref[...] = (
            jnp.dot(x_ref[...], w_ref[...], preferred_element_type=jnp.float32)
            + b_ref[...]
        )

    return pl.pallas_call(
        body,
        grid=(N // BR,),
        in_specs=[
            pl.BlockSpec((BR, D), lambda i: (i, 0)),
            pl.BlockSpec((D, D), lambda i: (0, 0)),
            pl.BlockSpec((1, D), lambda i: (0, 0)),
        ],
        out_specs=pl.BlockSpec((BR, D), lambda i: (i, 0)),
        out_shape=jax.ShapeDtypeStruct((N, D), jnp.float32),
    )(x, w, b)


def _make_tc_combine(relu):
    def body(pagg_ref, pdeg_ref, base_ref, wl_ref, o_ref):
        s = pagg_ref[0] + pagg_ref[1]
        dcol = pdeg_ref[0, :, 0:1] + pdeg_ref[1, :, 0:1]
        recip = 1.0 / jnp.maximum(dcol, 1.0)
        acc = (
            jnp.dot(s * recip, wl_ref[...], preferred_element_type=jnp.float32)
            + base_ref[...]
        )
        if relu:
            acc = jnp.maximum(acc, 0.0)
        o_ref[...] = acc

    def run(pagg, pdeg, base, wl):
        return pl.pallas_call(
            body,
            grid=(N // BR,),
            in_specs=[
                pl.BlockSpec((NC, BR, D), lambda i: (0, i, 0)),
                pl.BlockSpec((NC, BR, D), lambda i: (0, i, 0)),
                pl.BlockSpec((BR, D), lambda i: (i, 0)),
                pl.BlockSpec((D, D), lambda i: (0, 0)),
            ],
            out_specs=pl.BlockSpec((BR, D), lambda i: (i, 0)),
            out_shape=jax.ShapeDtypeStruct((N, D), jnp.float32),
        )(pagg, pdeg, base, wl)

    return run


_tc_combine_relu = _make_tc_combine(True)
_tc_combine_plain = _make_tc_combine(False)


def kernel(x, edge_index, W1_l, W1_r, b1, W2_l, W2_r, b2):
    src = edge_index[0].astype(jnp.int32).reshape(NW, NCHUNK, CH)
    dst = edge_index[1].astype(jnp.int32).reshape(NW, NCHUNK, CH)

    deg = _sc_deg(dst)

    agg1 = _sc_agg(x, src, dst)
    base1 = _tc_self(x, W1_r.T, b1.reshape(1, D))
    h = _tc_combine_relu(agg1, deg, base1, W1_l.T)

    agg2 = _sc_agg(h, src, dst)
    base2 = _tc_self(h, W2_r.T, b2.reshape(1, D))
    out = _tc_combine_plain(agg2, deg, base2, W2_l.T)
    return out


# speedup vs baseline: 8.0630x; 1.1790x over previous
import functools

import jax
import jax.numpy as jnp
from jax import lax
from jax.experimental import pallas as pl
from jax.experimental.pallas import tpu as pltpu
from jax.experimental.pallas import tpu_sc as plsc

N = 10000
E = 320000
D = 128

NC = 2
NS = 16
L = 16
NW = NC * NS
EPT = E // NW
CH = 40
NCHUNK = EPT // CH
K = 5
IB = 50
NBLK = NCHUNK // IB
NG = IB // K
RA = 624
TAIL = N - NS * RA
ZR = 16

BR = 400

_mesh = plsc.VectorSubcoreMesh(core_axis_name="c", subcore_axis_name="s")


@functools.partial(
    pl.kernel,
    mesh=_mesh,
    out_type=jax.ShapeDtypeStruct((NC, N, D), jnp.float32),
    scratch_types=[
        pltpu.VMEM((IB, CH), jnp.int32),
        pltpu.VMEM((IB, CH), jnp.int32),
        pltpu.VMEM((ZR, D), jnp.float32),
        pltpu.VMEM_SHARED((N, D), jnp.float32),
        pltpu.SemaphoreType.DMA,
        pltpu.SemaphoreType.DMA,
    ] + [pltpu.VMEM((CH, D), jnp.float32) for _ in range(K)],
)
def _sc_agg(feat_hbm, src_hbm, dst_hbm, agg_out, sidx, didx, zbuf, acc_sh,
            gsem, ssem, *rows):
    cid = lax.axis_index("c")
    sid = lax.axis_index("s")
    wid = cid * NS + sid
    rbase = sid * RA
    is_last = sid == NS - 1

    @pl.loop(0, ZR)
    def _(r):
        @pl.loop(0, D, step=L)
        def _(c0):
            zbuf[r, pl.ds(c0, L)] = jnp.zeros((L,), jnp.float32)

    @pl.loop(0, RA, step=ZR)
    def _(r0):
        pltpu.sync_copy(zbuf, acc_sh.at[pl.ds(rbase + r0, ZR)])

    @pl.when(is_last)
    def _():
        pltpu.sync_copy(zbuf.at[pl.ds(0, TAIL)],
                        acc_sh.at[pl.ds(NS * RA, TAIL)])

    plsc.subcore_barrier()

    @pl.loop(0, NBLK)
    def _(blk):
        pltpu.sync_copy(src_hbm.at[wid, blk], sidx)
        pltpu.sync_copy(dst_hbm.at[wid, blk], didx)

        @pl.loop(0, NG)
        def _(g):
            gat = [
                pltpu.async_copy(feat_hbm.at[sidx.at[g * K + b]], rows[b],
                                 gsem)
                for b in range(K)
            ]
            for h in gat:
                h.wait()
            sca = [
                pltpu.async_copy(rows[b], acc_sh.at[didx.at[g * K + b]],
                                 ssem, add=True)
                for b in range(K)
            ]
            for h in sca:
                h.wait()

    plsc.subcore_barrier()

    pltpu.sync_copy(acc_sh.at[pl.ds(rbase, RA)],
                    agg_out.at[cid, pl.ds(rbase, RA)])

    @pl.when(is_last)
    def _():
        pltpu.sync_copy(acc_sh.at[pl.ds(NS * RA, TAIL)],
                        agg_out.at[cid, pl.ds(NS * RA, TAIL)])


@functools.partial(
    pl.kernel,
    mesh=_mesh,
    out_type=jax.ShapeDtypeStruct((NC, N, D), jnp.float32),
    scratch_types=[
        pltpu.VMEM((IB, CH), jnp.int32),
        pltpu.VMEM((CH, D), jnp.float32),
        pltpu.VMEM((ZR, D), jnp.float32),
        pltpu.VMEM_SHARED((N, D), jnp.float32),
        pltpu.SemaphoreType.DMA,
    ],
)
def _sc_deg(dst_hbm, deg_out, didx, ones, dzbuf, deg_sh, ssem):
    cid = lax.axis_index("c")
    sid = lax.axis_index("s")
    wid = cid * NS + sid
    rbase = sid * RA
    is_last = sid == NS - 1

    @pl.loop(0, ZR)
    def _(r):
        @pl.loop(0, D, step=L)
        def _(c0):
            dzbuf[r, pl.ds(c0, L)] = jnp.zeros((L,), jnp.float32)

    @pl.loop(0, CH)
    def _(r):
        @pl.loop(0, D, step=L)
        def _(c0):
            ones[r, pl.ds(c0, L)] = jnp.ones((L,), jnp.float32)

    @pl.loop(0, RA, step=ZR)
    def _(r0):
        pltpu.sync_copy(dzbuf, deg_sh.at[pl.ds(rbase + r0, ZR)])

    @pl.when(is_last)
    def _():
        pltpu.sync_copy(dzbuf.at[pl.ds(0, TAIL)],
                        deg_sh.at[pl.ds(NS * RA, TAIL)])

    plsc.subcore_barrier()

    @pl.loop(0, NBLK)
    def _(blk):
        pltpu.sync_copy(dst_hbm.at[wid, blk], didx)

        @pl.loop(0, NG)
        def _(g):
            sca = [
                pltpu.async_copy(ones, deg_sh.at[didx.at[g * K + b]],
                                 ssem, add=True)
                for b in range(K)
            ]
            for h in sca:
                h.wait()

    plsc.subcore_barrier()

    pltpu.sync_copy(deg_sh.at[pl.ds(rbase, RA)],
                    deg_out.at[cid, pl.ds(rbase, RA)])

    @pl.when(is_last)
    def _():
        pltpu.sync_copy(deg_sh.at[pl.ds(NS * RA, TAIL)],
                        deg_out.at[cid, pl.ds(NS * RA, TAIL)])


def _tc_self(x, w, b):
    def body(x_ref, w_ref, b_ref, o_ref):
        o_ref[...] = (
            jnp.dot(x_ref[...], w_ref[...], preferred_element_type=jnp.float32)
            + b_ref[...]
        )

    return pl.pallas_call(
        body,
        grid=(N // BR,),
        in_specs=[
            pl.BlockSpec((BR, D), lambda i: (i, 0)),
            pl.BlockSpec((D, D), lambda i: (0, 0)),
            pl.BlockSpec((1, D), lambda i: (0, 0)),
        ],
        out_specs=pl.BlockSpec((BR, D), lambda i: (i, 0)),
        out_shape=jax.ShapeDtypeStruct((N, D), jnp.float32),
    )(x, w, b)


def _make_tc_combine(relu):
    def body(pagg_ref, pdeg_ref, base_ref, wl_ref, o_ref):
        s = pagg_ref[0] + pagg_ref[1]
        dcol = pdeg_ref[0, :, 0:1] + pdeg_ref[1, :, 0:1]
        recip = 1.0 / jnp.maximum(dcol, 1.0)
        acc = (
            jnp.dot(s * recip, wl_ref[...], preferred_element_type=jnp.float32)
            + base_ref[...]
        )
        if relu:
            acc = jnp.maximum(acc, 0.0)
        o_ref[...] = acc

    def run(pagg, pdeg, base, wl):
        return pl.pallas_call(
            body,
            grid=(N // BR,),
            in_specs=[
                pl.BlockSpec((NC, BR, D), lambda i: (0, i, 0)),
                pl.BlockSpec((NC, BR, D), lambda i: (0, i, 0)),
                pl.BlockSpec((BR, D), lambda i: (i, 0)),
                pl.BlockSpec((D, D), lambda i: (0, 0)),
            ],
            out_specs=pl.BlockSpec((BR, D), lambda i: (i, 0)),
            out_shape=jax.ShapeDtypeStruct((N, D), jnp.float32),
        )(pagg, pdeg, base, wl)

    return run


_tc_combine_relu = _make_tc_combine(True)
_tc_combine_plain = _make_tc_combine(False)


def kernel(x, edge_index, W1_l, W1_r, b1, W2_l, W2_r, b2):
    src = edge_index[0].astype(jnp.int32).reshape(NW, NBLK, IB, CH)
    dst = edge_index[1].astype(jnp.int32).reshape(NW, NBLK, IB, CH)

    deg = _sc_deg(dst)

    agg1 = _sc_agg(x, src, dst)
    base1 = _tc_self(x, W1_r.T, b1.reshape(1, D))
    h = _tc_combine_relu(agg1, deg, base1, W1_l.T)

    agg2 = _sc_agg(h, src, dst)
    base2 = _tc_self(h, W2_r.T, b2.reshape(1, D))
    out = _tc_combine_plain(agg2, deg, base2, W2_l.T)
    return out


# speedup vs baseline: 9.6273x; 1.1940x over previous
import functools

import jax
import jax.numpy as jnp
from jax import lax
from jax.experimental import pallas as pl
from jax.experimental.pallas import tpu as pltpu
from jax.experimental.pallas import tpu_sc as plsc

N = 10000
E = 320000
D = 128

NC = 2
NS = 16
L = 16
NW = NC * NS
EPT = E // NW
CH = 40
NCHUNK = EPT // CH
K = 5
IB = 50
NBLK = NCHUNK // IB
NG = IB // K
RA = 624
TAIL = N - NS * RA
ZR = 16

BR = 400

_mesh = plsc.VectorSubcoreMesh(core_axis_name="c", subcore_axis_name="s")


@functools.partial(
    pl.kernel,
    mesh=_mesh,
    out_type=jax.ShapeDtypeStruct((NC, N, D), jnp.float32),
    scratch_types=[
        pltpu.VMEM((IB, CH), jnp.int32),
        pltpu.VMEM((IB, CH), jnp.int32),
        pltpu.VMEM((ZR, D), jnp.float32),
        pltpu.VMEM_SHARED((N, D), jnp.float32),
    ] + [pltpu.SemaphoreType.DMA for _ in range(K)]
      + [pltpu.SemaphoreType.DMA for _ in range(K)]
      + [pltpu.VMEM((CH, D), jnp.float32) for _ in range(K)],
)
def _sc_agg(feat_hbm, src_hbm, dst_hbm, agg_out, sidx, didx, zbuf, acc_sh,
            *sems_rows):
    gsem = sems_rows[:K]
    ssem = sems_rows[K:2 * K]
    rows = sems_rows[2 * K:]
    cid = lax.axis_index("c")
    sid = lax.axis_index("s")
    wid = cid * NS + sid
    rbase = sid * RA
    is_last = sid == NS - 1

    @pl.loop(0, ZR)
    def _(r):
        @pl.loop(0, D, step=L)
        def _(c0):
            zbuf[r, pl.ds(c0, L)] = jnp.zeros((L,), jnp.float32)

    @pl.loop(0, RA, step=ZR)
    def _(r0):
        pltpu.sync_copy(zbuf, acc_sh.at[pl.ds(rbase + r0, ZR)])

    @pl.when(is_last)
    def _():
        pltpu.sync_copy(zbuf.at[pl.ds(0, TAIL)],
                        acc_sh.at[pl.ds(NS * RA, TAIL)])

    plsc.subcore_barrier()

    @pl.loop(0, NBLK)
    def _(blk):
        pltpu.sync_copy(src_hbm.at[wid, blk], sidx)
        pltpu.sync_copy(dst_hbm.at[wid, blk], didx)

        for b in range(K):
            pltpu.async_copy(feat_hbm.at[sidx.at[b]], rows[b], gsem[b])

        @pl.loop(0, NG)
        def _(g):
            sca = []
            for b in range(K):
                pltpu.make_async_copy(feat_hbm.at[sidx.at[g * K + b]],
                                      rows[b], gsem[b]).wait()
                sca.append(
                    pltpu.async_copy(rows[b], acc_sh.at[didx.at[g * K + b]],
                                     ssem[b], add=True))

            @pl.when(g < NG - 1)
            def _():
                for b in range(K):
                    sca[b].wait()
                    pltpu.async_copy(feat_hbm.at[sidx.at[(g + 1) * K + b]],
                                     rows[b], gsem[b])

        for b in range(K):
            pltpu.make_async_copy(rows[b],
                                  acc_sh.at[didx.at[(NG - 1) * K + b]],
                                  ssem[b]).wait()

    plsc.subcore_barrier()

    pltpu.sync_copy(acc_sh.at[pl.ds(rbase, RA)],
                    agg_out.at[cid, pl.ds(rbase, RA)])

    @pl.when(is_last)
    def _():
        pltpu.sync_copy(acc_sh.at[pl.ds(NS * RA, TAIL)],
                        agg_out.at[cid, pl.ds(NS * RA, TAIL)])


def _make_sc_deg(W):
    @functools.partial(
        pl.kernel,
        mesh=_mesh,
        out_type=jax.ShapeDtypeStruct((NC, N, W), jnp.float32),
        scratch_types=[
            pltpu.VMEM((IB, CH), jnp.int32),
            pltpu.VMEM((CH, W), jnp.float32),
            pltpu.VMEM((ZR, W), jnp.float32),
            pltpu.VMEM_SHARED((N, W), jnp.float32),
            pltpu.SemaphoreType.DMA,
        ],
    )
    def k(dst_hbm, deg_out, didx, ones, dzbuf, deg_sh, ssem):
        cid = lax.axis_index("c")
        sid = lax.axis_index("s")
        wid = cid * NS + sid
        rbase = sid * RA
        is_last = sid == NS - 1

        @pl.loop(0, ZR)
        def _(r):
            @pl.loop(0, W, step=L)
            def _(c0):
                dzbuf[r, pl.ds(c0, L)] = jnp.zeros((L,), jnp.float32)

        @pl.loop(0, CH)
        def _(r):
            @pl.loop(0, W, step=L)
            def _(c0):
                ones[r, pl.ds(c0, L)] = jnp.ones((L,), jnp.float32)

        @pl.loop(0, RA, step=ZR)
        def _(r0):
            pltpu.sync_copy(dzbuf, deg_sh.at[pl.ds(rbase + r0, ZR)])

        @pl.when(is_last)
        def _():
            pltpu.sync_copy(dzbuf.at[pl.ds(0, TAIL)],
                            deg_sh.at[pl.ds(NS * RA, TAIL)])

        plsc.subcore_barrier()

        @pl.loop(0, NBLK)
        def _(blk):
            pltpu.sync_copy(dst_hbm.at[wid, blk], didx)

            @pl.loop(0, NG)
            def _(g):
                sca = [
                    pltpu.async_copy(ones, deg_sh.at[didx.at[g * K + b]],
                                     ssem, add=True)
                    for b in range(K)
                ]
                for h in sca:
                    h.wait()

        plsc.subcore_barrier()

        pltpu.sync_copy(deg_sh.at[pl.ds(rbase, RA)],
                        deg_out.at[cid, pl.ds(rbase, RA)])

        @pl.when(is_last)
        def _():
            pltpu.sync_copy(deg_sh.at[pl.ds(NS * RA, TAIL)],
                            deg_out.at[cid, pl.ds(NS * RA, TAIL)])

    return k


DEGW = 128
_sc_deg = _make_sc_deg(DEGW)


def _make_tc_combine(relu):
    def body(pagg_ref, pdeg_ref, x_ref, wl_ref, wr_ref, b_ref, o_ref):
        s = pagg_ref[0] + pagg_ref[1]
        dcol = pdeg_ref[0, :, 0:1] + pdeg_ref[1, :, 0:1]
        recip = 1.0 / jnp.maximum(dcol, 1.0)
        acc = (
            jnp.dot(s * recip, wl_ref[...], preferred_element_type=jnp.float32)
            + jnp.dot(x_ref[...], wr_ref[...],
                      preferred_element_type=jnp.float32)
            + b_ref[...]
        )
        if relu:
            acc = jnp.maximum(acc, 0.0)
        o_ref[...] = acc

    def run(pagg, pdeg, x, wl, wr, b):
        return pl.pallas_call(
            body,
            grid=(N // BR,),
            in_specs=[
                pl.BlockSpec((NC, BR, D), lambda i: (0, i, 0)),
                pl.BlockSpec((NC, BR, DEGW), lambda i: (0, i, 0)),
                pl.BlockSpec((BR, D), lambda i: (i, 0)),
                pl.BlockSpec((D, D), lambda i: (0, 0)),
                pl.BlockSpec((D, D), lambda i: (0, 0)),
                pl.BlockSpec((1, D), lambda i: (0, 0)),
            ],
            out_specs=pl.BlockSpec((BR, D), lambda i: (i, 0)),
            out_shape=jax.ShapeDtypeStruct((N, D), jnp.float32),
        )(pagg, pdeg, x, wl, wr, b)

    return run


_tc_combine_relu = _make_tc_combine(True)
_tc_combine_plain = _make_tc_combine(False)


def kernel(x, edge_index, W1_l, W1_r, b1, W2_l, W2_r, b2):
    src = edge_index[0].astype(jnp.int32).reshape(NW, NBLK, IB, CH)
    dst = edge_index[1].astype(jnp.int32).reshape(NW, NBLK, IB, CH)

    deg = _sc_deg(dst)

    agg1 = _sc_agg(x, src, dst)
    h = _tc_combine_relu(agg1, deg, x, W1_l.T, W1_r.T, b1.reshape(1, D))

    agg2 = _sc_agg(h, src, dst)
    out = _tc_combine_plain(agg2, deg, h, W2_l.T, W2_r.T, b2.reshape(1, D))
    return out


# speedup vs baseline: 10.1239x; 1.0516x over previous
import functools

import jax
import jax.numpy as jnp
from jax import lax
from jax.experimental import pallas as pl
from jax.experimental.pallas import tpu as pltpu
from jax.experimental.pallas import tpu_sc as plsc

N = 10000
E = 320000
D = 128

NC = 2
NS = 16
L = 16
NW = NC * NS
EPT = E // NW
CH = 40
NCHUNK = EPT // CH
K = 5
IB = 50
NBLK = NCHUNK // IB
NG = IB // K
RA = 624
TAIL = N - NS * RA
ZR = 16

BR = 1000

_mesh = plsc.VectorSubcoreMesh(core_axis_name="c", subcore_axis_name="s")


def _make_sc_agg(with_deg):
    out_type = [jax.ShapeDtypeStruct((NC, N, D), jnp.float32)]
    scratch = [
        pltpu.VMEM((IB, CH), jnp.int32),
        pltpu.VMEM((IB, CH), jnp.int32),
        pltpu.VMEM((ZR, D), jnp.float32),
        pltpu.VMEM_SHARED((N, D), jnp.float32),
    ]
    if with_deg:
        out_type.append(jax.ShapeDtypeStruct((NC, N, D), jnp.float32))
        scratch.append(pltpu.VMEM((CH, D), jnp.float32))
    scratch += ([pltpu.SemaphoreType.DMA for _ in range(K)]
                + [pltpu.SemaphoreType.DMA for _ in range(K)]
                + [pltpu.VMEM((CH, D), jnp.float32) for _ in range(K)])

    @functools.partial(
        pl.kernel,
        mesh=_mesh,
        out_type=tuple(out_type) if with_deg else out_type[0],
        scratch_types=scratch,
    )
    def k(feat_hbm, src_hbm, dst_hbm, *refs):
        if with_deg:
            agg_out, deg_out, sidx, didx, zbuf, acc_sh, ones = refs[:7]
            rest = refs[7:]
        else:
            agg_out, sidx, didx, zbuf, acc_sh = refs[:5]
            deg_out = ones = None
            rest = refs[5:]
        gsem = rest[:K]
        ssem = rest[K:2 * K]
        rows = rest[2 * K:]
        cid = lax.axis_index("c")
        sid = lax.axis_index("s")
        wid = cid * NS + sid
        rbase = sid * RA
        is_last = sid == NS - 1

        @pl.loop(0, ZR)
        def _(r):
            @pl.loop(0, D, step=L)
            def _(c0):
                zbuf[r, pl.ds(c0, L)] = jnp.zeros((L,), jnp.float32)

        def zero_acc():
            @pl.loop(0, RA, step=ZR)
            def _(r0):
                pltpu.sync_copy(zbuf, acc_sh.at[pl.ds(rbase + r0, ZR)])

            @pl.when(is_last)
            def _():
                pltpu.sync_copy(zbuf.at[pl.ds(0, TAIL)],
                                acc_sh.at[pl.ds(NS * RA, TAIL)])

        def write_out(out):
            pltpu.sync_copy(acc_sh.at[pl.ds(rbase, RA)],
                            out.at[cid, pl.ds(rbase, RA)])

            @pl.when(is_last)
            def _():
                pltpu.sync_copy(acc_sh.at[pl.ds(NS * RA, TAIL)],
                                out.at[cid, pl.ds(NS * RA, TAIL)])

        zero_acc()

        if with_deg:
            @pl.loop(0, CH)
            def _(r):
                @pl.loop(0, D, step=L)
                def _(c0):
                    ones[r, pl.ds(c0, L)] = jnp.ones((L,), jnp.float32)

            plsc.subcore_barrier()

            @pl.loop(0, NBLK)
            def _(blk):
                pltpu.sync_copy(dst_hbm.at[wid, blk], didx)

                @pl.loop(0, NG)
                def _(g):
                    sca = [
                        pltpu.async_copy(ones,
                                         acc_sh.at[didx.at[g * K + b]],
                                         ssem[b], add=True)
                        for b in range(K)
                    ]
                    for h in sca:
                        h.wait()

            plsc.subcore_barrier()
            write_out(deg_out)
            zero_acc()

        plsc.subcore_barrier()

        @pl.loop(0, NBLK)
        def _(blk):
            pltpu.sync_copy(src_hbm.at[wid, blk], sidx)
            pltpu.sync_copy(dst_hbm.at[wid, blk], didx)

            for b in range(K):
                pltpu.async_copy(feat_hbm.at[sidx.at[b]], rows[b], gsem[b])

            @pl.loop(0, NG)
            def _(g):
                sca = []
                for b in range(K):
                    pltpu.make_async_copy(feat_hbm.at[sidx.at[g * K + b]],
                                          rows[b], gsem[b]).wait()
                    sca.append(
                        pltpu.async_copy(rows[b],
                                         acc_sh.at[didx.at[g * K + b]],
                                         ssem[b], add=True))

                @pl.when(g < NG - 1)
                def _():
                    for b in range(K):
                        sca[b].wait()
                        pltpu.async_copy(
                            feat_hbm.at[sidx.at[(g + 1) * K + b]],
                            rows[b], gsem[b])

            for b in range(K):
                pltpu.make_async_copy(rows[b],
                                      acc_sh.at[didx.at[(NG - 1) * K + b]],
                                      ssem[b]).wait()

        plsc.subcore_barrier()
        write_out(agg_out)

    return k


_sc_agg_deg = _make_sc_agg(True)
_sc_agg = _make_sc_agg(False)


def _make_tc_combine(relu):
    def body(pagg_ref, pdeg_ref, x_ref, wl_ref, wr_ref, b_ref, o_ref):
        s = pagg_ref[0] + pagg_ref[1]
        dcol = pdeg_ref[0, :, 0:1] + pdeg_ref[1, :, 0:1]
        recip = 1.0 / jnp.maximum(dcol, 1.0)
        acc = (
            jnp.dot(s * recip, wl_ref[...], preferred_element_type=jnp.float32)
            + jnp.dot(x_ref[...], wr_ref[...],
                      preferred_element_type=jnp.float32)
            + b_ref[...]
        )
        if relu:
            acc = jnp.maximum(acc, 0.0)
        o_ref[...] = acc

    def run(pagg, pdeg, x, wl, wr, b):
        return pl.pallas_call(
            body,
            grid=(N // BR,),
            in_specs=[
                pl.BlockSpec((NC, BR, D), lambda i: (0, i, 0)),
                pl.BlockSpec((NC, BR, D), lambda i: (0, i, 0)),
                pl.BlockSpec((BR, D), lambda i: (i, 0)),
                pl.BlockSpec((D, D), lambda i: (0, 0)),
                pl.BlockSpec((D, D), lambda i: (0, 0)),
                pl.BlockSpec((1, D), lambda i: (0, 0)),
            ],
            out_specs=pl.BlockSpec((BR, D), lambda i: (i, 0)),
            out_shape=jax.ShapeDtypeStruct((N, D), jnp.float32),
        )(pagg, pdeg, x, wl, wr, b)

    return run


_tc_combine_relu = _make_tc_combine(True)
_tc_combine_plain = _make_tc_combine(False)


def kernel(x, edge_index, W1_l, W1_r, b1, W2_l, W2_r, b2):
    src = edge_index[0].astype(jnp.int32).reshape(NW, NBLK, IB, CH)
    dst = edge_index[1].astype(jnp.int32).reshape(NW, NBLK, IB, CH)

    agg1, deg = _sc_agg_deg(x, src, dst)
    h = _tc_combine_relu(agg1, deg, x, W1_l.T, W1_r.T, b1.reshape(1, D))

    agg2 = _sc_agg(h, src, dst)
    out = _tc_combine_plain(agg2, deg, h, W2_l.T, W2_r.T, b2.reshape(1, D))
    return out


# speedup vs baseline: 10.1301x; 1.0006x over previous
import functools

import jax
import jax.numpy as jnp
from jax import lax
from jax.experimental import pallas as pl
from jax.experimental.pallas import tpu as pltpu
from jax.experimental.pallas import tpu_sc as plsc

N = 10000
E = 320000
D = 128

NC = 2
NS = 16
L = 16
NW = NC * NS
EPT = E // NW
CH = 40
NCHUNK = EPT // CH
K = 5
IB = 50
NBLK = NCHUNK // IB
NG = IB // K
RA = 624
TAIL = N - NS * RA
ZR = 16

BR = 1000

_mesh = plsc.VectorSubcoreMesh(core_axis_name="c", subcore_axis_name="s")


def _make_sc_agg(with_deg):
    out_type = [jax.ShapeDtypeStruct((NC, N, D), jnp.float32)]
    scratch = [
        pltpu.VMEM((IB, CH), jnp.int32),
        pltpu.VMEM((IB, CH), jnp.int32),
        pltpu.VMEM((ZR, D), jnp.float32),
        pltpu.VMEM_SHARED((N, D), jnp.float32),
    ]
    if with_deg:
        out_type.append(jax.ShapeDtypeStruct((NC, N, D), jnp.float32))
        scratch.append(pltpu.VMEM((CH, D), jnp.float32))
    scratch += ([pltpu.SemaphoreType.DMA for _ in range(K)]
                + [pltpu.SemaphoreType.DMA for _ in range(K)]
                + [pltpu.VMEM((CH, D), jnp.float32) for _ in range(K)])

    @functools.partial(
        pl.kernel,
        mesh=_mesh,
        out_type=tuple(out_type) if with_deg else out_type[0],
        scratch_types=scratch,
    )
    def k(feat_hbm, src_hbm, dst_hbm, *refs):
        if with_deg:
            agg_out, deg_out, sidx, didx, zbuf, acc_sh, ones = refs[:7]
            rest = refs[7:]
        else:
            agg_out, sidx, didx, zbuf, acc_sh = refs[:5]
            deg_out = ones = None
            rest = refs[5:]
        gsem = rest[:K]
        ssem = rest[K:2 * K]
        rows = rest[2 * K:]
        cid = lax.axis_index("c")
        sid = lax.axis_index("s")
        wid = cid * NS + sid
        rbase = sid * RA
        is_last = sid == NS - 1

        @pl.loop(0, ZR)
        def _(r):
            @pl.loop(0, D, step=L)
            def _(c0):
                zbuf[r, pl.ds(c0, L)] = jnp.zeros((L,), jnp.float32)

        def zero_acc():
            @pl.loop(0, RA, step=ZR)
            def _(r0):
                pltpu.sync_copy(zbuf, acc_sh.at[pl.ds(rbase + r0, ZR)])

            @pl.when(is_last)
            def _():
                pltpu.sync_copy(zbuf.at[pl.ds(0, TAIL)],
                                acc_sh.at[pl.ds(NS * RA, TAIL)])

        def write_out(out):
            pltpu.sync_copy(acc_sh.at[pl.ds(rbase, RA)],
                            out.at[cid, pl.ds(rbase, RA)])

            @pl.when(is_last)
            def _():
                pltpu.sync_copy(acc_sh.at[pl.ds(NS * RA, TAIL)],
                                out.at[cid, pl.ds(NS * RA, TAIL)])

        zero_acc()

        if with_deg:
            @pl.loop(0, CH)
            def _(r):
                @pl.loop(0, D, step=L)
                def _(c0):
                    ones[r, pl.ds(c0, L)] = jnp.ones((L,), jnp.float32)

            plsc.subcore_barrier()

            @pl.loop(0, NBLK)
            def _(blk):
                pltpu.sync_copy(dst_hbm.at[wid, blk], didx)

                @pl.loop(0, NG)
                def _(g):
                    for b in range(K):
                        pltpu.async_copy(ones,
                                         acc_sh.at[didx.at[g * K + b]],
                                         ssem[b], add=True)

                for b in range(K):
                    @pl.loop(0, NG)
                    def _(g):
                        pltpu.make_async_copy(ones, acc_sh.at[didx.at[0]],
                                              ssem[b]).wait()

            plsc.subcore_barrier()
            write_out(deg_out)
            zero_acc()

        plsc.subcore_barrier()

        @pl.loop(0, NBLK)
        def _(blk):
            pltpu.sync_copy(src_hbm.at[wid, blk], sidx)
            pltpu.sync_copy(dst_hbm.at[wid, blk], didx)

            for b in range(K):
                pltpu.async_copy(feat_hbm.at[sidx.at[b]], rows[b], gsem[b])

            @pl.loop(0, NG)
            def _(g):
                sca = []
                for b in range(K):
                    pltpu.make_async_copy(feat_hbm.at[sidx.at[g * K + b]],
                                          rows[b], gsem[b]).wait()
                    sca.append(
                        pltpu.async_copy(rows[b],
                                         acc_sh.at[didx.at[g * K + b]],
                                         ssem[b], add=True))

                @pl.when(g < NG - 1)
                def _():
                    for b in range(K):
                        sca[b].wait()
                        pltpu.async_copy(
                            feat_hbm.at[sidx.at[(g + 1) * K + b]],
                            rows[b], gsem[b])

            for b in range(K):
                pltpu.make_async_copy(rows[b],
                                      acc_sh.at[didx.at[(NG - 1) * K + b]],
                                      ssem[b]).wait()

        plsc.subcore_barrier()
        write_out(agg_out)

    return k


_sc_agg_deg = _make_sc_agg(True)
_sc_agg = _make_sc_agg(False)


def _make_tc_combine(relu):
    def body(pagg_ref, pdeg_ref, x_ref, wl_ref, wr_ref, b_ref, o_ref):
        s = pagg_ref[0] + pagg_ref[1]
        dcol = pdeg_ref[0, :, 0:1] + pdeg_ref[1, :, 0:1]
        recip = 1.0 / jnp.maximum(dcol, 1.0)
        acc = (
            jnp.dot(s * recip, wl_ref[...], preferred_element_type=jnp.float32)
            + jnp.dot(x_ref[...], wr_ref[...],
                      preferred_element_type=jnp.float32)
            + b_ref[...]
        )
        if relu:
            acc = jnp.maximum(acc, 0.0)
        o_ref[...] = acc

    def run(pagg, pdeg, x, wl, wr, b):
        return pl.pallas_call(
            body,
            grid=(N // BR,),
            in_specs=[
                pl.BlockSpec((NC, BR, D), lambda i: (0, i, 0)),
                pl.BlockSpec((NC, BR, D), lambda i: (0, i, 0)),
                pl.BlockSpec((BR, D), lambda i: (i, 0)),
                pl.BlockSpec((D, D), lambda i: (0, 0)),
                pl.BlockSpec((D, D), lambda i: (0, 0)),
                pl.BlockSpec((1, D), lambda i: (0, 0)),
            ],
            out_specs=pl.BlockSpec((BR, D), lambda i: (i, 0)),
            out_shape=jax.ShapeDtypeStruct((N, D), jnp.float32),
        )(pagg, pdeg, x, wl, wr, b)

    return run


_tc_combine_relu = _make_tc_combine(True)
_tc_combine_plain = _make_tc_combine(False)


def kernel(x, edge_index, W1_l, W1_r, b1, W2_l, W2_r, b2):
    src = edge_index[0].astype(jnp.int32).reshape(NW, NBLK, IB, CH)
    dst = edge_index[1].astype(jnp.int32).reshape(NW, NBLK, IB, CH)

    agg1, deg = _sc_agg_deg(x, src, dst)
    h = _tc_combine_relu(agg1, deg, x, W1_l.T, W1_r.T, b1.reshape(1, D))

    agg2 = _sc_agg(h, src, dst)
    out = _tc_combine_plain(agg2, deg, h, W2_l.T, W2_r.T, b2.reshape(1, D))
    return out
